# baseline (speedup 1.0000x reference)
import functools

import jax
import jax.numpy as jnp
from jax.experimental import pallas as pl
from jax.experimental.pallas import tpu as pltpu

B, N, M, D = 8, 2048, 2048, 64
TM = 512
J = M // TM


def _chamfer_kernel(x_ref, y_ref, acc_ref, rowmin_ref):
    j = pl.program_id(1)
    b = pl.program_id(0)

    x = x_ref[0]
    yt = y_ref[0]

    xy = jnp.dot(x, yt.T, preferred_element_type=jnp.float32)
    x2 = jnp.sum(x * x, axis=1, keepdims=True)
    y2 = jnp.sum(yt * yt, axis=1)[None, :]
    d = jnp.maximum(x2 + y2 - 2.0 * xy, 0.0)

    rowmin = jnp.min(d, axis=1)
    colmin = jnp.min(d, axis=0)

    @pl.when(j == 0)
    def _():
        rowmin_ref[...] = rowmin

    @pl.when(j != 0)
    def _():
        rowmin_ref[...] = jnp.minimum(rowmin_ref[...], rowmin)

    @pl.when((b == 0) & (j == 0))
    def _():
        acc_ref[0, 0] = 0.0

    acc_ref[0, 0] += jnp.sum(colmin) * (1.0 / (M * B))

    @pl.when(j == J - 1)
    def _():
        acc_ref[0, 0] += jnp.sum(rowmin_ref[...]) * (1.0 / (N * B))


@jax.jit
def kernel(x, y):
    acc = pl.pallas_call(
        _chamfer_kernel,
        grid=(B, J),
        in_specs=[
            pl.BlockSpec((1, N, D), lambda b, j: (b, 0, 0)),
            pl.BlockSpec((1, TM, D), lambda b, j: (b, j, 0)),
        ],
        out_specs=pl.BlockSpec(
            (1, 1), lambda b, j: (0, 0), memory_space=pltpu.SMEM),
        out_shape=jax.ShapeDtypeStruct((1, 1), jnp.float32),
        scratch_shapes=[pltpu.VMEM((N,), jnp.float32)],
    )(x, y)
    return acc[0, 0]
